# baseline (speedup 1.0000x reference)
import functools

import jax
import jax.numpy as jnp
from jax import lax
from jax.experimental import pallas as pl
from jax.experimental.pallas import tpu as pltpu
from jax.experimental.pallas import tpu_sc as plsc

NC = 2
NS = 16
LANES = 16
NW = NC * NS

NCOPY = 4
SEG = 320


def _body(L, D, ids_hbm, unfair_hbm, fair_hbm, pos_hbm, tm_hbm,
          out_hbm,
          ids_v, fid_v, fair_v, pos_v, stage_v, cnt_v,
          sem_cp, sem_ids, sem_g):
    wid = lax.axis_index("s") * NC + lax.axis_index("c")
    n_tok = ids_hbm.shape[0]
    per_w = n_tok // NW
    w0 = wid * per_w
    cp_words = per_w * D // NCOPY

    for s in range(NCOPY):
        pltpu.async_copy(
            unfair_hbm.at[pl.ds(w0 * D + s * cp_words, cp_words)],
            out_hbm.at[pl.ds(w0 * D + s * cp_words, cp_words)], sem_cp)

    pltpu.sync_copy(fair_hbm, fair_v)
    pltpu.sync_copy(pos_hbm, pos_v)
    pltpu.sync_copy(ids_hbm.at[pl.ds(w0, per_w)], ids_v)
    gcps = [
        pltpu.async_copy(tm_hbm.at[ids_v.at[pl.ds(o, 128)]],
                         fid_v.at[pl.ds(o, 128)], sem_g)
        for o in range(0, per_w, 128)
    ]
    for cp in gcps:
        cp.wait()
    for s in range(NCOPY):
        pltpu.make_async_copy(
            unfair_hbm.at[pl.ds(0, cp_words)],
            out_hbm.at[pl.ds(0, cp_words)], sem_cp).wait()

    lane_iota = lax.broadcasted_iota(jnp.int32, (LANES,), 0)
    zero16 = jnp.zeros((LANES,), jnp.int32)

    def or_tree(x):
        for sh in (1, 2, 4, 8):
            cnt_v[pl.ds(0, LANES)] = x
            x = x | plsc.load_gather(cnt_v, [(lane_iota + sh) & (LANES - 1)])
        return x

    def seg_body(s, _):
        t0 = s * SEG

        def acc_body(g, acc):
            return acc | fid_v[pl.ds(t0 + g * LANES, LANES)]

        acc = lax.fori_loop(0, SEG // LANES, acc_body, zero16)

        @pl.when(or_tree(acc)[0] != 0)
        def _dirty_seg():
            def group_body(g, _):
                fid16 = fid_v[pl.ds(t0 + g * LANES, LANES)]
                mask = fid16 != 0

                @pl.when(or_tree(fid16)[0] != 0)
                def _group():
                    tok0 = w0 + t0 + g * LANES
                    pos16 = (tok0 + lane_iota) % L
                    for c0 in range(D):
                        col = jnp.full((LANES,), c0, jnp.int32)
                        vals = (plsc.load_gather(fair_v, [fid16, col])
                                + plsc.load_gather(pos_v, [pos16, col]))
                        plsc.store_scatter(stage_v,
                                           [lane_iota * D + col], vals)
                    for k in range(LANES):
                        @pl.when(fid16[k] != 0)
                        def _row(k=k):
                            pltpu.sync_copy(
                                stage_v.at[pl.ds(k * D, D)],
                                out_hbm.at[pl.ds((tok0 + k) * D, D)])

                return 0

            lax.fori_loop(0, SEG // LANES, group_body, 0)

        return 0

    lax.fori_loop(0, per_w // SEG, seg_body, 0)


def kernel(input_ids, unfair_embeds, fair_table, pos_table, token_map):
    B, L = input_ids.shape
    D = unfair_embeds.shape[-1]
    N = B * L
    per_w = N // NW
    assert N % NW == 0 and per_w % SEG == 0 and per_w % 128 == 0
    assert (per_w * D) % NCOPY == 0

    ids_flat = input_ids.reshape(N)
    unfair = unfair_embeds.reshape(N * D)
    pos_sl = pos_table[:L]

    mesh = plsc.VectorSubcoreMesh(core_axis_name="c", subcore_axis_name="s",
                                  num_cores=NC, num_subcores=NS)
    kfn = pl.kernel(
        functools.partial(_body, L, D),
        out_type=jax.ShapeDtypeStruct((N * D,), jnp.float32),
        mesh=mesh,
        scratch_types=[
            pltpu.VMEM((per_w,), jnp.int32),
            pltpu.VMEM((per_w,), jnp.int32),
            pltpu.VMEM((fair_table.shape[0], D), jnp.float32),
            pltpu.VMEM((L, D), jnp.float32),
            pltpu.VMEM((LANES * D,), jnp.float32),
            pltpu.VMEM((128,), jnp.int32),
            pltpu.SemaphoreType.DMA,
            pltpu.SemaphoreType.DMA,
            pltpu.SemaphoreType.DMA,
        ],
        compiler_params=pltpu.CompilerParams(needs_layout_passes=False),
    )
    out = kfn(ids_flat, unfair, fair_table, pos_sl, token_map)
    return out.reshape(B, L, D)


# speedup vs baseline: 25.6996x; 25.6996x over previous
import functools

import jax
import jax.numpy as jnp
from jax import lax
from jax.experimental import pallas as pl
from jax.experimental.pallas import tpu as pltpu
from jax.experimental.pallas import tpu_sc as plsc

NC = 2
NS = 16
LANES = 16
NW = NC * NS

CHUNK = 320
GSEG = [(o, min(128, CHUNK - o)) for o in range(0, CHUNK, 128)]


def _body(L, D, ids_hbm, unfair_hbm, fair_hbm, pos_hbm, tm_hbm,
          out_hbm,
          ids_v0, ids_v1, fid_v0, fid_v1, fair_v, pos_v,
          stage_v, cnt_v, sh0, sh1,
          sem_in0, sem_in1, sem_out0, sem_out1,
          sem_ids0, sem_ids1, sem_g0, sem_g1):
    ids_v = (ids_v0, ids_v1)
    fid_v = (fid_v0, fid_v1)
    sh = (sh0, sh1)
    sem_in = (sem_in0, sem_in1)
    sem_out = (sem_out0, sem_out1)
    sem_ids = (sem_ids0, sem_ids1)
    sem_g = (sem_g0, sem_g1)

    sid = lax.axis_index("s")
    wid = sid * NC + lax.axis_index("c")
    per_w = ids_hbm.shape[0] // NW
    n_chunks = per_w // CHUNK
    cw = CHUNK * D
    sb = sid * cw

    pltpu.sync_copy(fair_hbm, fair_v)
    pltpu.sync_copy(pos_hbm, pos_v)

    lane_iota = lax.broadcasted_iota(jnp.int32, (LANES,), 0)

    def r0_of(c):
        return wid * per_w + c * CHUNK

    def issue_in(c, b):
        pltpu.async_copy(unfair_hbm.at[pl.ds(r0_of(c) * D, cw)],
                         sh[b].at[pl.ds(sb, cw)], sem_in[b])

    def wait_in(b):
        pltpu.make_async_copy(unfair_hbm.at[pl.ds(0, cw)],
                              sh[b].at[pl.ds(sb, cw)], sem_in[b]).wait()

    def issue_out(c, b):
        pltpu.async_copy(sh[b].at[pl.ds(sb, cw)],
                         out_hbm.at[pl.ds(r0_of(c) * D, cw)], sem_out[b])

    def wait_out(b):
        pltpu.make_async_copy(sh[b].at[pl.ds(sb, cw)],
                              out_hbm.at[pl.ds(0, cw)], sem_out[b]).wait()

    def issue_ids(c, b):
        pltpu.async_copy(ids_hbm.at[pl.ds(r0_of(c), CHUNK)],
                         ids_v[b], sem_ids[b])

    def wait_ids(b):
        pltpu.make_async_copy(ids_hbm.at[pl.ds(0, CHUNK)],
                              ids_v[b], sem_ids[b]).wait()

    def issue_gather(b):
        for o, w in GSEG:
            pltpu.async_copy(tm_hbm.at[ids_v[b].at[pl.ds(o, w)]],
                             fid_v[b].at[pl.ds(o, w)], sem_g[b])

    def wait_gather(b):
        for o, w in GSEG:
            pltpu.make_async_copy(tm_hbm.at[ids_v[b].at[pl.ds(o, w)]],
                                  fid_v[b].at[pl.ds(o, w)], sem_g[b]).wait()

    def or_tree(x):
        for sh_ in (1, 2, 4, 8):
            cnt_v[pl.ds(0, LANES)] = x
            x = x | plsc.load_gather(cnt_v, [(lane_iota + sh_) & (LANES - 1)])
        return x

    def fixup(a, r0):
        fid = fid_v[a]
        acc = fid[pl.ds(0, LANES)]
        for g in range(1, CHUNK // LANES):
            acc = acc | fid[pl.ds(g * LANES, LANES)]

        @pl.when(or_tree(acc)[0] != 0)
        def _dirty_chunk():
            def group_body(g, _):
                fid16 = fid[pl.ds(g * LANES, LANES)]

                @pl.when(or_tree(fid16)[0] != 0)
                def _group():
                    pos16 = (r0 + g * LANES + lane_iota) % L
                    for c0 in range(D):
                        col = jnp.full((LANES,), c0, jnp.int32)
                        vals = (plsc.load_gather(fair_v, [fid16, col])
                                + plsc.load_gather(pos_v, [pos16, col]))
                        plsc.store_scatter(stage_v, [lane_iota * D + col],
                                           vals)
                    for k in range(LANES):
                        @pl.when(fid16[k] != 0)
                        def _row(k=k):
                            pltpu.sync_copy(
                                stage_v.at[pl.ds(k * D, D)],
                                sh[a].at[pl.ds(sb + (g * LANES + k) * D, D)])

                return 0

            lax.fori_loop(0, CHUNK // LANES, group_body, 0)

    issue_in(0, 0)
    issue_ids(0, 0)
    wait_ids(0)
    issue_gather(0)

    def pair_body(p, _):
        for par in range(2):
            c = p * 2 + par
            a, b = par, 1 - par

            @pl.when(c + 1 < n_chunks)
            def _prefetch():
                @pl.when(c > 0)
                def _():
                    wait_out(b)
                issue_in(c + 1, b)
                issue_ids(c + 1, b)

            wait_in(a)
            wait_gather(a)
            fixup(a, r0_of(c))

            @pl.when(c + 1 < n_chunks)
            def _next_gather():
                wait_ids(b)
                issue_gather(b)

            issue_out(c, a)
        return 0

    lax.fori_loop(0, n_chunks // 2, pair_body, 0)

    wait_out(0)
    wait_out(1)


def kernel(input_ids, unfair_embeds, fair_table, pos_table, token_map):
    B, L = input_ids.shape
    D = unfair_embeds.shape[-1]
    N = B * L
    assert N % (NW * CHUNK) == 0 and (N // (NW * CHUNK)) % 2 == 0 and D == 128

    ids_flat = input_ids.reshape(N)
    unfair = unfair_embeds.reshape(N * D)
    pos_sl = pos_table[:L]

    mesh = plsc.VectorSubcoreMesh(core_axis_name="c", subcore_axis_name="s",
                                  num_cores=NC, num_subcores=NS)
    kfn = pl.kernel(
        functools.partial(_body, L, D),
        out_type=jax.ShapeDtypeStruct((N * D,), jnp.float32),
        mesh=mesh,
        scratch_types=[
            pltpu.VMEM((CHUNK,), jnp.int32),
            pltpu.VMEM((CHUNK,), jnp.int32),
            pltpu.VMEM((CHUNK,), jnp.int32),
            pltpu.VMEM((CHUNK,), jnp.int32),
            pltpu.VMEM((fair_table.shape[0], D), jnp.float32),
            pltpu.VMEM((L, D), jnp.float32),
            pltpu.VMEM((LANES * D,), jnp.float32),
            pltpu.VMEM((128,), jnp.int32),
            pltpu.VMEM_SHARED((NS * CHUNK * D,), jnp.float32),
            pltpu.VMEM_SHARED((NS * CHUNK * D,), jnp.float32),
        ] + [pltpu.SemaphoreType.DMA] * 8,
        compiler_params=pltpu.CompilerParams(needs_layout_passes=False),
    )
    out = kfn(ids_flat, unfair, fair_table, pos_sl, token_map)
    return out.reshape(B, L, D)
